# baseline (speedup 1.0000x reference)
import functools

import jax
import jax.numpy as jnp
from jax.experimental import pallas as pl
from jax.experimental.pallas import tpu as pltpu

BV = 800
NQ = 4


def _mm_kernel(n_steps, emb_ref, x_ref, out_hbm, scratch, sems):
    i = pl.program_id(0)
    h = BV // NQ

    def copy(step, q, slot):
        return pltpu.make_async_copy(
            scratch.at[slot, pl.ds(q * h, h), :],
            out_hbm.at[pl.ds(step * BV + q * h, h), :],
            sems.at[slot, q],
        )

    @pl.when(i >= 2)
    def _():
        for q in range(NQ):
            copy(i - 2, q, jax.lax.rem(i, 2)).wait()

    slot = jax.lax.rem(i, 2)
    scratch[slot] = jax.lax.dot_general(
        emb_ref[...], x_ref[...],
        dimension_numbers=(((1,), (1,)), ((), ())),
        preferred_element_type=jnp.float32,
    )
    for q in range(NQ):
        copy(i, q, slot).start()

    @pl.when(i == n_steps - 1)
    def _():
        for q in range(NQ):
            copy(i - 1, q, jax.lax.rem(i - 1, 2)).wait()
        for q in range(NQ):
            copy(i, q, jax.lax.rem(i, 2)).wait()


@jax.jit
def kernel(x, emb_table):
    m, k = x.shape
    vocab, _ = emb_table.shape
    n_steps = vocab // BV
    out_t = pl.pallas_call(
        functools.partial(_mm_kernel, n_steps),
        grid=(n_steps,),
        in_specs=[
            pl.BlockSpec((BV, k), lambda j: (j, 0)),
            pl.BlockSpec((m, k), lambda j: (0, 0)),
        ],
        out_specs=pl.BlockSpec(memory_space=pl.ANY),
        out_shape=jax.ShapeDtypeStruct((vocab, m), jnp.float32),
        scratch_shapes=[
            pltpu.VMEM((2, BV, m), jnp.float32),
            pltpu.SemaphoreType.DMA((2, NQ)),
        ],
        compiler_params=pltpu.CompilerParams(
            dimension_semantics=("arbitrary",),
        ),
    )(emb_table, x)
    return out_t.T


# speedup vs baseline: 1.0113x; 1.0113x over previous
import jax
import jax.numpy as jnp
from jax.experimental import pallas as pl
from jax.experimental.pallas import tpu as pltpu

BV = 1280


def _matmul_kernel(emb_ref, x_ref, out_ref):
    out_ref[...] = jax.lax.dot_general(
        emb_ref[...], x_ref[...],
        dimension_numbers=(((1,), (1,)), ((), ())),
        preferred_element_type=jnp.float32,
    )


@jax.jit
def kernel(x, emb_table):
    m, k = x.shape
    vocab, _ = emb_table.shape
    grid = (pl.cdiv(vocab, BV),)
    out_t = pl.pallas_call(
        _matmul_kernel,
        grid=grid,
        in_specs=[
            pl.BlockSpec((BV, k), lambda j: (j, 0)),
            pl.BlockSpec((m, k), lambda j: (0, 0)),
        ],
        out_specs=pl.BlockSpec((BV, m), lambda j: (j, 0)),
        out_shape=jax.ShapeDtypeStruct((vocab, m), jnp.float32),
        compiler_params=pltpu.CompilerParams(
            dimension_semantics=("parallel",),
        ),
    )(emb_table, x)
    return out_t.T
